# baseline (speedup 1.0000x reference)
import functools

import jax
import jax.numpy as jnp
from jax import lax
from jax.experimental import pallas as pl
from jax.experimental.pallas import tpu as pltpu
from jax.experimental.pallas import tpu_sc as plsc

R = 128
C = 32768
K = 64
NW = 32
ROWS_PER_W = R // NW
NVREG = C // 16


def _sc_body(o_hbm, t_hbm, out_hbm, o_buf, t_buf, d_buf, outv, sem0, sem1):
    cid = lax.axis_index("c")
    sid = lax.axis_index("s")
    wid = sid * 2 + cid
    row0 = wid * ROWS_PER_W

    lane = lax.iota(jnp.int32, 16)
    inf16 = jnp.full((16,), jnp.inf, jnp.float32)
    zero16f = jnp.zeros((16,), jnp.float32)
    zero16i = jnp.zeros((16,), jnp.int32)
    one16i = jnp.ones((16,), jnp.int32)
    sixteen16 = jnp.full((16,), 16, jnp.int32)
    lane0 = lane == 0
    perms = [lane ^ sh for sh in (1, 2, 4, 8)]

    total_vec = zero16f
    cp_o = pltpu.async_copy(o_hbm.at[row0], o_buf, sem0)
    cp_t = pltpu.async_copy(t_hbm.at[row0], t_buf, sem1)
    for r in range(ROWS_PER_W):
        cp_o.wait()
        cp_t.wait()

        def pass1(i, mins):
            m0, m1, m2, m3 = mins
            base = i * 64
            new = []
            for q, m in enumerate((m0, m1, m2, m3)):
                o = o_buf[pl.ds(base + q * 16, 16)]
                t = t_buf[pl.ds(base + q * 16, 16)]
                d = o - t
                d = d * d
                d_buf[pl.ds(base + q * 16, 16)] = d
                new.append(jnp.minimum(m, d))
            return tuple(new)

        m0, m1, m2, m3 = plsc.parallel_loop(
            0, NVREG // 4, unroll=2,
            carry=(inf16, inf16, inf16, inf16))(pass1)
        u_bound = jnp.maximum(jnp.maximum(m0, m1), jnp.maximum(m2, m3))
        for p in perms:
            u_bound = jnp.maximum(
                u_bound, u_bound.at[p].get(mode="promise_in_bounds"))

        if r + 1 < ROWS_PER_W:
            cp_o = pltpu.async_copy(o_hbm.at[row0 + r + 1], o_buf, sem0)
            cp_t = pltpu.async_copy(t_hbm.at[row0 + r + 1], t_buf, sem1)

        def stage_a(i, off):
            base = i * 64
            anyvs = []
            incs = []
            ds = []
            for q in range(4):
                d = d_buf[pl.ds(base + q * 16, 16)]
                pc = plsc.all_reduce_population_count(d <= u_bound)
                anyv = pc > 0
                anyvs.append(anyv)
                incs.append(jnp.where(anyv, sixteen16, zero16i))
                ds.append(d)
            off_lane = off + lane
            p0 = zero16i
            p1 = incs[0]
            p2 = p1 + incs[1]
            p3 = p2 + incs[2]
            for q, part in enumerate((p0, p1, p2, p3)):
                plsc.store_scatter(
                    d_buf, [off_lane + part], ds[q], mask=anyvs[q])
            return off + (p3 + incs[3])

        offa = plsc.parallel_loop(
            0, NVREG // 4, unroll=2, carry=zero16i)(stage_a)
        na = jnp.max(offa)
        for k in range(4):
            plsc.store_scatter(d_buf, [offa + (k * 16) + lane], inf16)

        def stage_b(i, off):
            base = i * 64
            for q in range(4):
                d = d_buf[pl.ds(base + q * 16, 16)]
                msk = d <= u_bound
                mi = jnp.where(msk, one16i, zero16i)
                pos = off + jnp.cumsum(mi) - 1
                plsc.store_scatter(d_buf, [pos], d, mask=msk)
                off = off + plsc.all_reduce_population_count(msk)
            return off

        offb = plsc.parallel_loop(
            0, (na + 63) // 64, unroll=4, carry=zero16i)(stage_b)
        ncand = jnp.max(offb)
        nv4 = (ncand + 63) // 64
        for k in range(4):
            plsc.store_scatter(d_buf, [offb + (k * 16) + lane], inf16)

        def bstep(b, prefix):
            tst = prefix | jnp.left_shift(one16i, 30 - b)

            def cl(j, cnt):
                base = j * 64
                for q in range(4):
                    d = d_buf[pl.ds(base + q * 16, 16)]
                    bits = lax.bitcast_convert_type(d, jnp.int32)
                    cnt = cnt + plsc.all_reduce_population_count(bits < tst)
                return cnt

            cnt = lax.fori_loop(0, nv4, cl, zero16i)
            return jnp.where(cnt >= K, prefix, tst)

        vbits = lax.fori_loop(0, 31, bstep, zero16i)

        def fin(j, carry):
            cv, sv = carry
            base = j * 64
            for q in range(4):
                d = d_buf[pl.ds(base + q * 16, 16)]
                bits = lax.bitcast_convert_type(d, jnp.int32)
                ltm = bits < vbits
                cv = cv + plsc.all_reduce_population_count(ltm)
                sv = sv + jnp.where(ltm, d, zero16f)
            return cv, sv

        cv, sv = lax.fori_loop(0, nv4, fin, (zero16i, zero16f))
        vval = lax.bitcast_convert_type(vbits, jnp.float32)
        fill = (jnp.float32(K) - cv.astype(jnp.float32)) * vval
        total_vec = total_vec + sv + jnp.where(lane0, fill, zero16f)

    outv[...] = total_vec
    pltpu.sync_copy(outv, out_hbm.at[wid])


def kernel(output, target):
    mesh = plsc.VectorSubcoreMesh(core_axis_name="c", subcore_axis_name="s")
    run = functools.partial(
        pl.kernel,
        mesh=mesh,
        out_type=jax.ShapeDtypeStruct((NW, 16), jnp.float32),
        scratch_types=[
            pltpu.VMEM((C,), jnp.float32),
            pltpu.VMEM((C,), jnp.float32),
            pltpu.VMEM((C + 64,), jnp.float32),
            pltpu.VMEM((16,), jnp.float32),
            pltpu.SemaphoreType.DMA,
            pltpu.SemaphoreType.DMA,
        ],
        compiler_params=pltpu.CompilerParams(needs_layout_passes=False),
    )(_sc_body)
    partials = run(output, target)
    return jnp.sum(partials) / jnp.float32(R * K)


# speedup vs baseline: 1.0252x; 1.0252x over previous
import functools

import jax
import jax.numpy as jnp
from jax import lax
from jax.experimental import pallas as pl
from jax.experimental.pallas import tpu as pltpu
from jax.experimental.pallas import tpu_sc as plsc

R = 128
C = 32768
K = 64
NW = 32
ROWS_PER_W = R // NW
NVREG = C // 16


def _sc_body(o_hbm, t_hbm, out_hbm, o_buf, t_buf, d_buf, outv, sem0, sem1):
    cid = lax.axis_index("c")
    sid = lax.axis_index("s")
    wid = sid * 2 + cid
    row0 = wid * ROWS_PER_W

    lane = lax.iota(jnp.int32, 16)
    inf16 = jnp.full((16,), jnp.inf, jnp.float32)
    zero16f = jnp.zeros((16,), jnp.float32)
    zero16i = jnp.zeros((16,), jnp.int32)
    one16i = jnp.ones((16,), jnp.int32)
    sixteen16 = jnp.full((16,), 16, jnp.int32)
    lane0 = lane == 0
    perms = [lane ^ sh for sh in (1, 2, 4, 8)]

    total_vec = zero16f
    cp_o = pltpu.async_copy(o_hbm.at[row0], o_buf, sem0)
    cp_t = pltpu.async_copy(t_hbm.at[row0], t_buf, sem1)
    for r in range(ROWS_PER_W):
        cp_o.wait()
        cp_t.wait()

        def pass1(i, mins):
            m0, m1, m2, m3 = mins
            base = i * 64
            new = []
            for q, m in enumerate((m0, m1, m2, m3)):
                o = o_buf[pl.ds(base + q * 16, 16)]
                t = t_buf[pl.ds(base + q * 16, 16)]
                d = o - t
                d = d * d
                d_buf[pl.ds(base + q * 16, 16)] = d
                new.append(jnp.minimum(m, d))
            return tuple(new)

        m0, m1, m2, m3 = plsc.parallel_loop(
            0, NVREG // 4, unroll=2,
            carry=(inf16, inf16, inf16, inf16))(pass1)
        u_bound = jnp.maximum(jnp.maximum(m0, m1), jnp.maximum(m2, m3))
        for p in perms:
            u_bound = jnp.maximum(
                u_bound, u_bound.at[p].get(mode="promise_in_bounds"))

        if r + 1 < ROWS_PER_W:
            cp_o = pltpu.async_copy(o_hbm.at[row0 + r + 1], o_buf, sem0)
            cp_t = pltpu.async_copy(t_hbm.at[row0 + r + 1], t_buf, sem1)

        def stage_a(i, off):
            base = i * 64
            anyvs = []
            incs = []
            ds = []
            for q in range(4):
                d = d_buf[pl.ds(base + q * 16, 16)]
                pc = plsc.all_reduce_population_count(d <= u_bound)
                anyv = pc > 0
                anyvs.append(anyv)
                incs.append(jnp.where(anyv, sixteen16, zero16i))
                ds.append(d)
            off_lane = off + lane
            p0 = zero16i
            p1 = incs[0]
            p2 = p1 + incs[1]
            p3 = p2 + incs[2]
            for q, part in enumerate((p0, p1, p2, p3)):
                plsc.store_scatter(
                    d_buf, [off_lane + part], ds[q], mask=anyvs[q])
            return off + (p3 + incs[3])

        offa = plsc.parallel_loop(
            0, NVREG // 4, unroll=2, carry=zero16i)(stage_a)
        na = jnp.max(offa)
        for k in range(4):
            plsc.store_scatter(d_buf, [offa + (k * 16) + lane], inf16)

        def stage_b(i, off):
            base = i * 64
            for q in range(4):
                d = d_buf[pl.ds(base + q * 16, 16)]
                msk = d <= u_bound
                mi = jnp.where(msk, one16i, zero16i)
                pos = off + jnp.cumsum(mi) - 1
                plsc.store_scatter(d_buf, [pos], d, mask=msk)
                off = off + plsc.all_reduce_population_count(msk)
            return off

        offb = plsc.parallel_loop(
            0, (na + 63) // 64, unroll=2, carry=zero16i)(stage_b)
        ncand = jnp.max(offb)
        nv4 = (ncand + 63) // 64
        for k in range(4):
            plsc.store_scatter(d_buf, [offb + (k * 16) + lane], inf16)

        def bstep(b, prefix):
            tst = prefix | jnp.left_shift(one16i, 30 - b)

            def cl(j, cnt):
                base = j * 64
                for q in range(4):
                    d = d_buf[pl.ds(base + q * 16, 16)]
                    bits = lax.bitcast_convert_type(d, jnp.int32)
                    cnt = cnt + plsc.all_reduce_population_count(bits < tst)
                return cnt

            cnt = lax.fori_loop(0, nv4, cl, zero16i)
            return jnp.where(cnt >= K, prefix, tst)

        vbits = lax.fori_loop(0, 31, bstep, zero16i)

        def fin(j, carry):
            cv, sv = carry
            base = j * 64
            for q in range(4):
                d = d_buf[pl.ds(base + q * 16, 16)]
                bits = lax.bitcast_convert_type(d, jnp.int32)
                ltm = bits < vbits
                cv = cv + plsc.all_reduce_population_count(ltm)
                sv = sv + jnp.where(ltm, d, zero16f)
            return cv, sv

        cv, sv = plsc.parallel_loop(
            0, nv4, unroll=2, carry=(zero16i, zero16f))(fin)
        vval = lax.bitcast_convert_type(vbits, jnp.float32)
        fill = (jnp.float32(K) - cv.astype(jnp.float32)) * vval
        total_vec = total_vec + sv + jnp.where(lane0, fill, zero16f)

    outv[...] = total_vec
    pltpu.sync_copy(outv, out_hbm.at[wid])


def kernel(output, target):
    mesh = plsc.VectorSubcoreMesh(core_axis_name="c", subcore_axis_name="s")
    run = functools.partial(
        pl.kernel,
        mesh=mesh,
        out_type=jax.ShapeDtypeStruct((NW, 16), jnp.float32),
        scratch_types=[
            pltpu.VMEM((C,), jnp.float32),
            pltpu.VMEM((C,), jnp.float32),
            pltpu.VMEM((C + 64,), jnp.float32),
            pltpu.VMEM((16,), jnp.float32),
            pltpu.SemaphoreType.DMA,
            pltpu.SemaphoreType.DMA,
        ],
        compiler_params=pltpu.CompilerParams(needs_layout_passes=False),
    )(_sc_body)
    partials = run(output, target)
    return jnp.sum(partials) / jnp.float32(R * K)
